# baseline (speedup 1.0000x reference)
import functools

import jax
import jax.numpy as jnp
from jax.experimental import pallas as pl
from jax.experimental.pallas import tpu as pltpu
from jax.experimental.shard_map import shard_map
from jax.sharding import Mesh, PartitionSpec as P

IN_F = 2048
OUT_F = 2048
RDIM = 32
KSEL = 8
SCALE = 64.0 / 32.0


def _fused_kernel(x_ref, a_ref, b_ref, d_ref, o_ref):
    x_blk = x_ref[...]
    a = a_ref[...]
    b = b_ref[...]
    d = d_ref[...]

    y = jax.lax.dot_general(
        x_blk, a, (((1,), (1,)), ((), ())),
        preferred_element_type=jnp.float32)
    yb = jnp.abs(y + d)

    keys = jnp.transpose(yb).view(jnp.int32)
    row = jax.lax.broadcasted_iota(jnp.int32, (RDIM, keys.shape[1]), 0)
    rank = jnp.zeros(keys.shape, jnp.int32)
    for j in range(RDIM):
        kj = jnp.zeros_like(keys) + keys[j:j + 1, :]
        gt = (kj > keys).astype(jnp.int32)
        ge = (kj >= keys).astype(jnp.int32)
        rank = rank + jnp.where(row > j, ge, gt)
    mask_t = jnp.where(rank < KSEL, jnp.float32(SCALE), jnp.float32(0.0))
    mask = jnp.transpose(mask_t)

    act = (y * mask).astype(jnp.bfloat16)
    out = jax.lax.dot_general(
        act, b, (((1,), (1,)), ((), ())),
        preferred_element_type=jnp.float32)
    o_ref[...] = out


def _fused_call(x, A, b_bf, d2, bl):
    n_tokens = x.shape[0]
    grid = (n_tokens // bl,)
    return pl.pallas_call(
        _fused_kernel,
        grid=grid,
        in_specs=[
            pl.BlockSpec((bl, IN_F), lambda i: (i, 0)),
            pl.BlockSpec((RDIM, IN_F), lambda i: (0, 0)),
            pl.BlockSpec((OUT_F, RDIM), lambda i: (0, 0)),
            pl.BlockSpec((1, RDIM), lambda i: (0, 0)),
        ],
        out_specs=pl.BlockSpec((bl, OUT_F), lambda i: (i, 0)),
        out_shape=jax.ShapeDtypeStruct((n_tokens, OUT_F), jnp.float32),
        compiler_params=pltpu.CompilerParams(
            dimension_semantics=("parallel",)),
    )(x, A, b_bf, d2)


@jax.jit
def kernel(x, A, B, d):
    d2 = d.reshape(1, RDIM)
    b_bf = B.astype(jnp.bfloat16)
    devs = jax.devices()
    n_dev = len(devs)
    if n_dev > 1 and x.shape[0] % (n_dev * 1024) == 0:
        mesh = Mesh(devs, ("dp",))
        fn = shard_map(
            functools.partial(_fused_call, bl=1024),
            mesh=mesh,
            in_specs=(P("dp"), P(), P(), P()),
            out_specs=P("dp"),
            check_rep=False,
        )
        return fn(x, A, b_bf, d2)
    return _fused_call(x, A, b_bf, d2, bl=1024)


# speedup vs baseline: 6.5030x; 6.5030x over previous
import jax
import jax.numpy as jnp
from jax.experimental import pallas as pl
from jax.experimental.pallas import tpu as pltpu

IN_F = 2048
OUT_F = 2048
RDIM = 32
KSEL = 8
SCALE = 64.0 / 32.0
N_OUT_SPLIT = 2


def _fused_kernel(x_ref, a_ref, b_ref, d_ref, o_ref, act_ref):
    @pl.when(pl.program_id(1) == 0)
    def _routing():
        x_blk = x_ref[...]
        a = a_ref[...]
        d = d_ref[...]

        y = jax.lax.dot_general(
            x_blk, a, (((1,), (1,)), ((), ())),
            preferred_element_type=jnp.float32)
        yb = jnp.abs(y + d)

        keys = jnp.transpose(yb).view(jnp.int32)
        row = jax.lax.broadcasted_iota(jnp.int32, (RDIM, keys.shape[1]), 0)
        rank = jnp.zeros(keys.shape, jnp.int32)
        for j in range(RDIM):
            kj = jnp.zeros_like(keys) + keys[j:j + 1, :]
            gt = (kj > keys).astype(jnp.int32)
            ge = (kj >= keys).astype(jnp.int32)
            rank = rank + jnp.where(row > j, ge, gt)
        mask_t = jnp.where(rank < KSEL, jnp.float32(SCALE), jnp.float32(0.0))
        mask = jnp.transpose(mask_t)
        act_ref[...] = (y * mask).astype(jnp.bfloat16)

    out = jax.lax.dot_general(
        act_ref[...], b_ref[...], (((1,), (1,)), ((), ())),
        preferred_element_type=jnp.float32)
    o_ref[...] = out


def _fused_call(x, A, b_bf, d2, bl):
    n_tokens = x.shape[0]
    grid = (n_tokens // bl, N_OUT_SPLIT)
    out_w = OUT_F // N_OUT_SPLIT
    return pl.pallas_call(
        _fused_kernel,
        grid=grid,
        in_specs=[
            pl.BlockSpec((bl, IN_F), lambda i, j: (i, 0)),
            pl.BlockSpec((RDIM, IN_F), lambda i, j: (0, 0)),
            pl.BlockSpec((out_w, RDIM), lambda i, j: (j, 0)),
            pl.BlockSpec((1, RDIM), lambda i, j: (0, 0)),
        ],
        out_specs=pl.BlockSpec((bl, out_w), lambda i, j: (i, j)),
        out_shape=jax.ShapeDtypeStruct((n_tokens, OUT_F), jnp.float32),
        scratch_shapes=[pltpu.VMEM((bl, RDIM), jnp.bfloat16)],
        compiler_params=pltpu.CompilerParams(
            dimension_semantics=("parallel", "arbitrary")),
    )(x, A, b_bf, d2)


@jax.jit
def kernel(x, A, B, d):
    d2 = d.reshape(1, RDIM)
    b_bf = B.astype(jnp.bfloat16)
    return _fused_call(x, A, b_bf, d2, bl=2048)


# speedup vs baseline: 8.7691x; 1.3485x over previous
import jax
import jax.numpy as jnp
from jax.experimental import pallas as pl
from jax.experimental.pallas import tpu as pltpu

IN_F = 2048
OUT_F = 2048
RDIM = 32
KSEL = 8
SCALE = 64.0 / 32.0


def _routing_matmuls(x_blk, a, b, d):
    y = jax.lax.dot_general(
        x_blk, a, (((1,), (1,)), ((), ())),
        preferred_element_type=jnp.float32)
    yb = jnp.abs(y + d)

    keys = jnp.transpose(yb).view(jnp.int32)
    row = jax.lax.broadcasted_iota(jnp.int32, (RDIM, keys.shape[1]), 0)
    rank = jnp.zeros(keys.shape, jnp.int32)
    for j in range(RDIM):
        kj = jnp.zeros_like(keys) + keys[j:j + 1, :]
        gt = (kj > keys).astype(jnp.int32)
        ge = (kj >= keys).astype(jnp.int32)
        rank = rank + jnp.where(row > j, ge, gt)
    mask_t = jnp.where(rank < KSEL, jnp.float32(SCALE), jnp.float32(0.0))
    mask = jnp.transpose(mask_t)

    act = (y * mask).astype(jnp.bfloat16)
    return jax.lax.dot_general(
        act, b, (((1,), (1,)), ((), ())),
        preferred_element_type=jnp.float32)


def _fused_kernel(x_ref, a_ref, b_ref, d_ref, o_ref):
    a = a_ref[...]
    b = b_ref[...]
    d = d_ref[...]
    bl = x_ref.shape[0]
    h = bl // 2
    o_ref[0:h, :] = _routing_matmuls(x_ref[0:h, :], a, b, d)
    o_ref[h:bl, :] = _routing_matmuls(x_ref[h:bl, :], a, b, d)


def _fused_call(x, A, b_bf, d2, bl):
    n_tokens = x.shape[0]
    grid = (n_tokens // bl,)
    return pl.pallas_call(
        _fused_kernel,
        grid=grid,
        in_specs=[
            pl.BlockSpec((bl, IN_F), lambda i: (i, 0)),
            pl.BlockSpec((RDIM, IN_F), lambda i: (0, 0)),
            pl.BlockSpec((OUT_F, RDIM), lambda i: (0, 0)),
            pl.BlockSpec((1, RDIM), lambda i: (0, 0)),
        ],
        out_specs=pl.BlockSpec((bl, OUT_F), lambda i: (i, 0)),
        out_shape=jax.ShapeDtypeStruct((n_tokens, OUT_F), jnp.float32),
        compiler_params=pltpu.CompilerParams(
            dimension_semantics=("parallel",)),
    )(x, A, b_bf, d2)


@jax.jit
def kernel(x, A, B, d):
    d2 = d.reshape(1, RDIM)
    b_bf = B.astype(jnp.bfloat16)
    return _fused_call(x, A, b_bf, d2, bl=1024)


# speedup vs baseline: 9.1675x; 1.0454x over previous
import jax
import jax.numpy as jnp
from jax.experimental import pallas as pl
from jax.experimental.pallas import tpu as pltpu

IN_F = 2048
OUT_F = 2048
RDIM = 32
KSEL = 8
SCALE = 64.0 / 32.0


def _routing_matmuls(x_blk, a, b, d):
    y = jax.lax.dot_general(
        x_blk, a, (((1,), (1,)), ((), ())),
        preferred_element_type=jnp.float32)
    yb = jnp.abs(y + d)

    keys = jnp.transpose(yb).view(jnp.int32)
    row = jax.lax.broadcasted_iota(jnp.int32, (RDIM, keys.shape[1]), 0)
    rank = jnp.zeros(keys.shape, jnp.int32)
    for j in range(RDIM):
        kj = jnp.zeros_like(keys) + keys[j:j + 1, :]
        gt = (kj > keys).astype(jnp.int32)
        ge = (kj >= keys).astype(jnp.int32)
        rank = rank + jnp.where(row > j, ge, gt)
    mask_t = jnp.where(rank < KSEL, jnp.float32(SCALE), jnp.float32(0.0))
    mask = jnp.transpose(mask_t)

    act = (y * mask).astype(jnp.bfloat16)
    return jax.lax.dot_general(
        act, b, (((1,), (1,)), ((), ())),
        preferred_element_type=jnp.float32)


def _fused_kernel(x_ref, a_ref, b_ref, d_ref, o_ref):
    a = a_ref[...]
    b = b_ref[...]
    d = d_ref[...]
    bl = x_ref.shape[0]
    h = bl // 4
    for c in range(4):
        o_ref[c * h:(c + 1) * h, :] = _routing_matmuls(
            x_ref[c * h:(c + 1) * h, :], a, b, d)


def _fused_call(x, A, b_bf, d2, bl):
    n_tokens = x.shape[0]
    grid = (n_tokens // bl,)
    return pl.pallas_call(
        _fused_kernel,
        grid=grid,
        in_specs=[
            pl.BlockSpec((bl, IN_F), lambda i: (i, 0)),
            pl.BlockSpec((RDIM, IN_F), lambda i: (0, 0)),
            pl.BlockSpec((OUT_F, RDIM), lambda i: (0, 0)),
            pl.BlockSpec((1, RDIM), lambda i: (0, 0)),
        ],
        out_specs=pl.BlockSpec((bl, OUT_F), lambda i: (i, 0)),
        out_shape=jax.ShapeDtypeStruct((n_tokens, OUT_F), jnp.float32),
        compiler_params=pltpu.CompilerParams(
            dimension_semantics=("parallel",)),
    )(x, A, b_bf, d2)


@jax.jit
def kernel(x, A, B, d):
    d2 = d.reshape(1, RDIM)
    b_bf = B.astype(jnp.bfloat16)
    return _fused_call(x, A, b_bf, d2, bl=1024)


# speedup vs baseline: 9.3331x; 1.0181x over previous
import jax
import jax.numpy as jnp
from jax.experimental import pallas as pl
from jax.experimental.pallas import tpu as pltpu

IN_F = 2048
OUT_F = 2048
RDIM = 32
KSEL = 8
SCALE = 64.0 / 32.0


def _routing_matmuls(x_blk, a, b, d):
    y = jax.lax.dot_general(
        x_blk, a, (((1,), (1,)), ((), ())),
        preferred_element_type=jnp.float32)
    yb = jnp.abs(y + d)

    keys = jnp.transpose(yb).view(jnp.int32)
    row = jax.lax.broadcasted_iota(jnp.int32, (RDIM, keys.shape[1]), 0)
    rank = jnp.zeros(keys.shape, jnp.int32)
    for j in range(RDIM):
        kj = jnp.zeros_like(keys) + keys[j:j + 1, :]
        gt = (kj > keys).astype(jnp.int32)
        ge = (kj >= keys).astype(jnp.int32)
        rank = rank + jnp.where(row > j, ge, gt)
    mask_t = jnp.where(rank < KSEL, jnp.float32(SCALE), jnp.float32(0.0))
    mask = jnp.transpose(mask_t)

    act = (y * mask).astype(jnp.bfloat16)
    return jax.lax.dot_general(
        act, b, (((1,), (1,)), ((), ())),
        preferred_element_type=jnp.float32)


def _fused_kernel(x_ref, a_ref, b_ref, d_ref, o_ref):
    a = a_ref[...]
    b = b_ref[...]
    d = d_ref[...]
    bl = x_ref.shape[0]
    h = bl // 8
    for c in range(8):
        o_ref[c * h:(c + 1) * h, :] = _routing_matmuls(
            x_ref[c * h:(c + 1) * h, :], a, b, d)


def _fused_call(x, A, b_bf, d2, bl):
    n_tokens = x.shape[0]
    grid = (n_tokens // bl,)
    return pl.pallas_call(
        _fused_kernel,
        grid=grid,
        in_specs=[
            pl.BlockSpec((bl, IN_F), lambda i: (i, 0)),
            pl.BlockSpec((RDIM, IN_F), lambda i: (0, 0)),
            pl.BlockSpec((OUT_F, RDIM), lambda i: (0, 0)),
            pl.BlockSpec((1, RDIM), lambda i: (0, 0)),
        ],
        out_specs=pl.BlockSpec((bl, OUT_F), lambda i: (i, 0)),
        out_shape=jax.ShapeDtypeStruct((n_tokens, OUT_F), jnp.float32),
        compiler_params=pltpu.CompilerParams(
            dimension_semantics=("parallel",)),
    )(x, A, b_bf, d2)


@jax.jit
def kernel(x, A, B, d):
    d2 = d.reshape(1, RDIM)
    b_bf = B.astype(jnp.bfloat16)
    return _fused_call(x, A, b_bf, d2, bl=1024)
